# baseline (speedup 1.0000x reference)
import functools

import jax
import jax.numpy as jnp
from jax import lax
from jax.experimental import pallas as pl
from jax.experimental.pallas import tpu as pltpu
from jax.experimental.pallas import tpu_sc as plsc

T = 4096
D = 1024
H = 1024
E = 8
P = 2 * T
BMG = 256
NBE = P // BMG + E
PP = NBE * BMG

NC, NS, NL = 2, 16, 16
NW = NC * NS

_DN = (((1,), (1,)), ((), ()))


def _route_body(x_ref, wr_ref, bias_ref, dest_ref, wp_ref, be_ref):
    xs = x_ref[...]
    logits = lax.dot_general(xs, wr_ref[...], _DN, preferred_element_type=jnp.float32)
    scores = jax.nn.sigmoid(logits + bias_ref[...][None, :])

    m1 = jnp.max(scores, axis=-1)
    i1 = jnp.argmax(scores, axis=-1).astype(jnp.int32)
    cols = lax.broadcasted_iota(jnp.int32, scores.shape, 1)
    masked = jnp.where(cols == i1[:, None], -jnp.inf, scores)
    m2 = jnp.max(masked, axis=-1)
    i2 = jnp.argmax(masked, axis=-1).astype(jnp.int32)
    denom = m1 + m2 + 1e-6
    w1 = m1 / denom
    w2 = m2 / denom

    ep = jnp.concatenate([i1, i2]).reshape(P // 128, 128)

    r128 = lax.broadcasted_iota(jnp.int32, (128, 128), 0)
    c128 = lax.broadcasted_iota(jnp.int32, (128, 128), 1)
    upper_incl = (r128 <= c128).astype(jnp.float32)
    nr = P // 128
    rr = lax.broadcasted_iota(jnp.int32, (nr, nr), 0)
    cc = lax.broadcasted_iota(jnp.int32, (nr, nr), 1)
    lower_strict = (cc < rr).astype(jnp.float32)

    dn_std = (((1,), (0,)), ((), ()))
    dest_f = jnp.zeros((nr, 128), jnp.float32)
    off = 0.0
    ends = []
    for e in range(E):
        m = (ep == e).astype(jnp.float32)
        incl = lax.dot_general(m, upper_incl, dn_std, preferred_element_type=jnp.float32)
        rowtot = incl[:, 127:128]
        row_off = lax.dot_general(lower_strict, rowtot, dn_std, preferred_element_type=jnp.float32)
        rank = incl - m + row_off
        cnt = jnp.sum(m)
        padded = jnp.ceil(cnt / BMG) * BMG
        dest_f = dest_f + m * (off + rank)
        off = off + padded
        ends.append(off)

    dest_ref[...] = dest_f.astype(jnp.int32)
    wp_ref[...] = jnp.stack([w1, w2])

    bi = lax.broadcasted_iota(jnp.int32, (8, 8), 0) * 8 + lax.broadcasted_iota(
        jnp.int32, (8, 8), 1
    )
    blk_start = bi.astype(jnp.float32) * BMG
    be = jnp.zeros((8, 8), jnp.int32)
    for e in range(E):
        be = be + (blk_start >= ends[e]).astype(jnp.int32)
    be_ref[...] = jnp.minimum(be, E - 1)


def _route(x_flat, Wr, expert_bias):
    dest, wp, be = pl.pallas_call(
        _route_body,
        out_shape=[
            jax.ShapeDtypeStruct((P // 128, 128), jnp.int32),
            jax.ShapeDtypeStruct((2, T), jnp.float32),
            jax.ShapeDtypeStruct((8, 8), jnp.int32),
        ],
    )(x_flat, Wr, expert_bias)
    return dest.reshape(P), wp, be.reshape(64)[:NBE]


def _dispatch_body(dest_hbm, x_hbm, xs_hbm, di0, di1, st0, st1, rw0, rw1, gs0, gs1, ss0, ss1):
    wid = lax.axis_index("s") * NC + lax.axis_index("c")
    base = wid * (P // NW)
    di = [di0, di1]
    st = [st0, st1]
    rw = [rw0, rw1]
    gs = [gs0, gs1]
    ss = [ss0, ss1]

    def start_gather(j):
        b = j % 2
        p0 = base + j * 32
        pltpu.sync_copy(dest_hbm.at[pl.ds(p0, 32)], di[b])
        for i in range(2):
            v = lax.iota(jnp.int32, 16) + (p0 + i * 16)
            v = v - jnp.where(v >= T, T, 0)
            st[b][pl.ds(i * 16, 16)] = v
        return pltpu.async_copy(x_hbm.at[st[b]], rw[b], gs[b])

    gd = [start_gather(0), start_gather(1)]
    sd = [None, None]
    for j in range(8):
        b = j % 2
        gd[b].wait()
        sd[b] = pltpu.async_copy(rw[b], xs_hbm.at[di[b]], ss[b])
        if j + 2 < 8:
            sd[b].wait()
            gd[b] = start_gather(j + 2)
    sd[0].wait()
    sd[1].wait()


def _dispatch(dest, x_flat):
    mesh = plsc.VectorSubcoreMesh(core_axis_name="c", subcore_axis_name="s")
    f = pl.kernel(
        _dispatch_body,
        out_type=jax.ShapeDtypeStruct((PP, D // 2), jnp.int32),
        mesh=mesh,
        scratch_types=[
            pltpu.VMEM((32,), jnp.int32),
            pltpu.VMEM((32,), jnp.int32),
            pltpu.VMEM((32,), jnp.int32),
            pltpu.VMEM((32,), jnp.int32),
            pltpu.VMEM((32, D // 2), jnp.int32),
            pltpu.VMEM((32, D // 2), jnp.int32),
            pltpu.SemaphoreType.DMA,
            pltpu.SemaphoreType.DMA,
            pltpu.SemaphoreType.DMA,
            pltpu.SemaphoreType.DMA,
        ],
    )
    return f(dest, x_flat)


def _swiglu_body(x_ref, wg_ref, wu_ref, wd_ref, o_ref):
    xb = x_ref[...]
    g = lax.dot_general(xb, wg_ref[...], _DN, preferred_element_type=jnp.float32)
    u = lax.dot_general(xb, wu_ref[...], _DN, preferred_element_type=jnp.float32)
    h = ((g * jax.nn.sigmoid(g)) * u).astype(x_ref.dtype)
    o_ref[...] = lax.dot_general(h, wd_ref[...], _DN, preferred_element_type=jnp.float32)


def _shared_expert(x_flat, Sg, Su, Sd):
    BM = 1024
    return pl.pallas_call(
        _swiglu_body,
        grid=(T // BM,),
        in_specs=[
            pl.BlockSpec((BM, D), lambda b: (b, 0)),
            pl.BlockSpec((H, D), lambda b: (0, 0)),
            pl.BlockSpec((H, D), lambda b: (0, 0)),
            pl.BlockSpec((D, H), lambda b: (0, 0)),
        ],
        out_specs=pl.BlockSpec((BM, D), lambda b: (b, 0)),
        out_shape=jax.ShapeDtypeStruct((T, D), jnp.float32),
    )(x_flat, Sg, Su, Sd)


def _grouped_body(be_ref, xs_ref, wg_ref, wu_ref, wd_ref, o_ref):
    _swiglu_body(xs_ref, wg_ref.at[0], wu_ref.at[0], wd_ref.at[0], o_ref)


def _grouped_gemm(be, Xs, Wg, Wu, Wd):
    grid_spec = pltpu.PrefetchScalarGridSpec(
        num_scalar_prefetch=1,
        grid=(NBE,),
        in_specs=[
            pl.BlockSpec((BMG, D), lambda b, be_ref: (b, 0)),
            pl.BlockSpec((1, H, D), lambda b, be_ref: (be_ref[b], 0, 0)),
            pl.BlockSpec((1, H, D), lambda b, be_ref: (be_ref[b], 0, 0)),
            pl.BlockSpec((1, D, H), lambda b, be_ref: (be_ref[b], 0, 0)),
        ],
        out_specs=pl.BlockSpec((BMG, D), lambda b, be_ref: (b, 0)),
    )
    return pl.pallas_call(
        _grouped_body,
        grid_spec=grid_spec,
        out_shape=jax.ShapeDtypeStruct((PP, D), jnp.float32),
    )(be, Xs, Wg, Wu, Wd)


def _combine_body(
    yex_hbm, ysh_hbm, dest_hbm, wp_hbm, out_hbm,
    da0, da1, db0, db1, w1v, w2v, wsp1, wsp2,
    r1a, r1b, r2a, r2b, aca, acb,
    s1a, s1b, s2a, s2b, sha, shb, soa, sob,
):
    wid = lax.axis_index("s") * NC + lax.axis_index("c")
    bt = wid * (T // NW)
    da = [da0, da1]
    db = [db0, db1]
    r1 = [r1a, r1b]
    r2 = [r2a, r2b]
    ac = [aca, acb]
    s1 = [s1a, s1b]
    s2 = [s2a, s2b]
    sh = [sha, shb]
    so = [soa, sob]
    pltpu.sync_copy(wp_hbm.at[pl.ds(bt, 128)], w1v)
    pltpu.sync_copy(wp_hbm.at[pl.ds(T + bt, 128)], w2v)

    def start_chunk(c):
        b = c % 2
        t0 = bt + c * 16
        pltpu.sync_copy(dest_hbm.at[pl.ds(t0, 16)], da[b])
        pltpu.sync_copy(dest_hbm.at[pl.ds(T + t0, 16)], db[b])
        return (
            pltpu.async_copy(yex_hbm.at[da[b]], r1[b], s1[b]),
            pltpu.async_copy(yex_hbm.at[db[b]], r2[b], s2[b]),
            pltpu.async_copy(ysh_hbm.at[pl.ds(t0, 16)], ac[b], sh[b]),
        )

    descs = [start_chunk(0), start_chunk(1)]
    wdesc = [None, None]
    for c in range(8):
        b = c % 2
        for d in descs[b]:
            d.wait()

        wv1 = w1v[pl.ds(c * 16, 16)]
        wv2 = w2v[pl.ds(c * 16, 16)]
        for r16 in range(16):
            wsp1[r16, :] = jnp.full((NL,), wv1[r16], jnp.float32)
            wsp2[r16, :] = jnp.full((NL,), wv2[r16], jnp.float32)

        def row_body(r, carry):
            w1s = wsp1[r, :]
            w2s = wsp2[r, :]
            for v in range(D // NL):
                sl = pl.ds(v * NL, NL)
                ac[b][r, sl] = ac[b][r, sl] + w1s * r1[b][r, sl] + w2s * r2[b][r, sl]
            return carry

        lax.fori_loop(0, 16, row_body, 0)
        wdesc[b] = pltpu.async_copy(ac[b], out_hbm.at[pl.ds(bt + c * 16, 16)], so[b])
        if c + 2 < 8:
            wdesc[b].wait()
            descs[b] = start_chunk(c + 2)
    wdesc[0].wait()
    wdesc[1].wait()


def _combine(Yex, Ysh, dest, wp_flat):
    mesh = plsc.VectorSubcoreMesh(core_axis_name="c", subcore_axis_name="s")
    f = pl.kernel(
        _combine_body,
        out_type=jax.ShapeDtypeStruct((T, D), jnp.float32),
        mesh=mesh,
        scratch_types=[
            pltpu.VMEM((16,), jnp.int32),
            pltpu.VMEM((16,), jnp.int32),
            pltpu.VMEM((16,), jnp.int32),
            pltpu.VMEM((16,), jnp.int32),
            pltpu.VMEM((128,), jnp.float32),
            pltpu.VMEM((128,), jnp.float32),
            pltpu.VMEM((16, NL), jnp.float32),
            pltpu.VMEM((16, NL), jnp.float32),
            pltpu.VMEM((16, D), jnp.float32),
            pltpu.VMEM((16, D), jnp.float32),
            pltpu.VMEM((16, D), jnp.float32),
            pltpu.VMEM((16, D), jnp.float32),
            pltpu.VMEM((16, D), jnp.float32),
            pltpu.VMEM((16, D), jnp.float32),
            pltpu.SemaphoreType.DMA,
            pltpu.SemaphoreType.DMA,
            pltpu.SemaphoreType.DMA,
            pltpu.SemaphoreType.DMA,
            pltpu.SemaphoreType.DMA,
            pltpu.SemaphoreType.DMA,
            pltpu.SemaphoreType.DMA,
            pltpu.SemaphoreType.DMA,
        ],
    )
    return f(Yex, Ysh, dest, wp_flat)


def kernel(x, Wr, Wg, Wu, Wd, Sg, Su, Sd, expert_bias):
    bsz, seqlen, dim = x.shape
    x_flat = x.reshape(bsz * seqlen, dim)

    dest, wp, be = _route(x_flat, Wr, expert_bias)
    x16 = x_flat.astype(jnp.bfloat16)
    x16_packed = lax.bitcast_convert_type(x16.reshape(T, D // 2, 2), jnp.int32)
    Xs_packed = _dispatch(dest, x16_packed)
    Xs = lax.bitcast_convert_type(Xs_packed, jnp.bfloat16).reshape(PP, D)
    Ysh = _shared_expert(
        x16, Sg.astype(jnp.bfloat16), Su.astype(jnp.bfloat16), Sd.astype(jnp.bfloat16)
    )
    Yex = _grouped_gemm(
        be, Xs, Wg.astype(jnp.bfloat16), Wu.astype(jnp.bfloat16), Wd.astype(jnp.bfloat16)
    )
    out = _combine(Yex, Ysh, dest, wp.reshape(P))
    return out.reshape(bsz, seqlen, dim)


# speedup vs baseline: 2.5361x; 2.5361x over previous
import functools

import jax
import jax.numpy as jnp
from jax import lax
from jax.experimental import pallas as pl
from jax.experimental.pallas import tpu as pltpu
from jax.experimental.pallas import tpu_sc as plsc

T = 4096
D = 1024
H = 1024
E = 8
P = 2 * T
BMG = 256
NBE = P // BMG + E
PP = NBE * BMG

NC, NS, NL = 2, 16, 16
NW = NC * NS

_DN = (((1,), (1,)), ((), ()))


def _route_body(x_ref, wr_ref, bias_ref, dest_ref, wp_ref, be_ref):
    xs = x_ref[...]
    logits = lax.dot_general(xs, wr_ref[...], _DN, preferred_element_type=jnp.float32)
    scores = jax.nn.sigmoid(logits + bias_ref[...][None, :])

    m1 = jnp.max(scores, axis=-1)
    i1 = jnp.argmax(scores, axis=-1).astype(jnp.int32)
    cols = lax.broadcasted_iota(jnp.int32, scores.shape, 1)
    masked = jnp.where(cols == i1[:, None], -jnp.inf, scores)
    m2 = jnp.max(masked, axis=-1)
    i2 = jnp.argmax(masked, axis=-1).astype(jnp.int32)
    denom = m1 + m2 + 1e-6
    w1 = m1 / denom
    w2 = m2 / denom

    ep = jnp.concatenate([i1, i2]).reshape(P // 128, 128)

    r128 = lax.broadcasted_iota(jnp.int32, (128, 128), 0)
    c128 = lax.broadcasted_iota(jnp.int32, (128, 128), 1)
    upper_incl = (r128 <= c128).astype(jnp.float32)
    nr = P // 128
    rr = lax.broadcasted_iota(jnp.int32, (nr, nr), 0)
    cc = lax.broadcasted_iota(jnp.int32, (nr, nr), 1)
    lower_strict = (cc < rr).astype(jnp.float32)

    dn_std = (((1,), (0,)), ((), ()))
    dest_f = jnp.zeros((nr, 128), jnp.float32)
    off = 0.0
    ends = []
    for e in range(E):
        m = (ep == e).astype(jnp.float32)
        incl = lax.dot_general(m, upper_incl, dn_std, preferred_element_type=jnp.float32)
        rowtot = incl[:, 127:128]
        row_off = lax.dot_general(lower_strict, rowtot, dn_std, preferred_element_type=jnp.float32)
        rank = incl - m + row_off
        cnt = jnp.sum(m)
        padded = jnp.ceil(cnt / BMG) * BMG
        dest_f = dest_f + m * (off + rank)
        off = off + padded
        ends.append(off)

    dest_ref[...] = dest_f.astype(jnp.int32)
    wp_ref[...] = jnp.stack([w1, w2])

    bi = lax.broadcasted_iota(jnp.int32, (8, 8), 0) * 8 + lax.broadcasted_iota(
        jnp.int32, (8, 8), 1
    )
    blk_start = bi.astype(jnp.float32) * BMG
    be = jnp.zeros((8, 8), jnp.int32)
    for e in range(E):
        be = be + (blk_start >= ends[e]).astype(jnp.int32)
    be_ref[...] = jnp.minimum(be, E - 1)


def _route(x_flat, Wr, expert_bias):
    dest, wp, be = pl.pallas_call(
        _route_body,
        out_shape=[
            jax.ShapeDtypeStruct((P // 128, 128), jnp.int32),
            jax.ShapeDtypeStruct((2, T), jnp.float32),
            jax.ShapeDtypeStruct((8, 8), jnp.int32),
        ],
    )(x_flat, Wr, expert_bias)
    return dest.reshape(P), wp, be.reshape(64)[:NBE]


def _dispatch_body(dest_hbm, x_hbm, xs_hbm, di0, di1, st0, st1, rw0, rw1, gs0, gs1, ss0, ss1):
    wid = lax.axis_index("s") * NC + lax.axis_index("c")
    base = wid * (P // NW)
    di = [di0, di1]
    st = [st0, st1]
    rw = [rw0, rw1]
    gs = [gs0, gs1]
    ss = [ss0, ss1]

    def start_gather(j):
        b = j % 2
        p0 = base + j * 32
        pltpu.sync_copy(dest_hbm.at[pl.ds(p0, 32)], di[b])
        for i in range(2):
            v = lax.iota(jnp.int32, 16) + (p0 + i * 16)
            v = v - jnp.where(v >= T, T, 0)
            st[b][pl.ds(i * 16, 16)] = v
        return pltpu.async_copy(x_hbm.at[st[b]], rw[b], gs[b])

    gd = [start_gather(0), start_gather(1)]
    sd = [None, None]
    for j in range(8):
        b = j % 2
        gd[b].wait()
        sd[b] = pltpu.async_copy(rw[b], xs_hbm.at[di[b]], ss[b])
        if j + 2 < 8:
            sd[b].wait()
            gd[b] = start_gather(j + 2)
    sd[0].wait()
    sd[1].wait()


def _dispatch(dest, x_flat):
    mesh = plsc.VectorSubcoreMesh(core_axis_name="c", subcore_axis_name="s")
    f = pl.kernel(
        _dispatch_body,
        out_type=jax.ShapeDtypeStruct((PP, D), jnp.float32),
        mesh=mesh,
        scratch_types=[
            pltpu.VMEM((32,), jnp.int32),
            pltpu.VMEM((32,), jnp.int32),
            pltpu.VMEM((32,), jnp.int32),
            pltpu.VMEM((32,), jnp.int32),
            pltpu.VMEM((32, D), jnp.float32),
            pltpu.VMEM((32, D), jnp.float32),
            pltpu.SemaphoreType.DMA,
            pltpu.SemaphoreType.DMA,
            pltpu.SemaphoreType.DMA,
            pltpu.SemaphoreType.DMA,
        ],
    )
    return f(dest, x_flat)


def _swiglu_body(x_ref, wg_ref, wu_ref, wd_ref, o_ref):
    xb = x_ref[...]
    g = lax.dot_general(xb, wg_ref[...], _DN, preferred_element_type=jnp.float32)
    u = lax.dot_general(xb, wu_ref[...], _DN, preferred_element_type=jnp.float32)
    h = ((g * jax.nn.sigmoid(g)) * u).astype(x_ref.dtype)
    o_ref[...] = lax.dot_general(h, wd_ref[...], _DN, preferred_element_type=jnp.float32)


def _shared_expert(x_flat, Sg, Su, Sd):
    BM = 1024
    return pl.pallas_call(
        _swiglu_body,
        grid=(T // BM,),
        in_specs=[
            pl.BlockSpec((BM, D), lambda b: (b, 0)),
            pl.BlockSpec((H, D), lambda b: (0, 0)),
            pl.BlockSpec((H, D), lambda b: (0, 0)),
            pl.BlockSpec((D, H), lambda b: (0, 0)),
        ],
        out_specs=pl.BlockSpec((BM, D), lambda b: (b, 0)),
        out_shape=jax.ShapeDtypeStruct((T, D), jnp.float32),
    )(x_flat, Sg, Su, Sd)


def _grouped_body(be_ref, xs_ref, wg_ref, wu_ref, wd_ref, o_ref):
    _swiglu_body(xs_ref, wg_ref.at[0], wu_ref.at[0], wd_ref.at[0], o_ref)


def _grouped_gemm(be, Xs, Wg, Wu, Wd):
    grid_spec = pltpu.PrefetchScalarGridSpec(
        num_scalar_prefetch=1,
        grid=(NBE,),
        in_specs=[
            pl.BlockSpec((BMG, D), lambda b, be_ref: (b, 0)),
            pl.BlockSpec((1, H, D), lambda b, be_ref: (be_ref[b], 0, 0)),
            pl.BlockSpec((1, H, D), lambda b, be_ref: (be_ref[b], 0, 0)),
            pl.BlockSpec((1, D, H), lambda b, be_ref: (be_ref[b], 0, 0)),
        ],
        out_specs=pl.BlockSpec((BMG, D), lambda b, be_ref: (b, 0)),
    )
    return pl.pallas_call(
        _grouped_body,
        grid_spec=grid_spec,
        out_shape=jax.ShapeDtypeStruct((PP, D), jnp.float32),
    )(be, Xs, Wg, Wu, Wd)


def _combine_body(
    yex_hbm, ysh_hbm, dest_hbm, wp_hbm, out_hbm,
    da0, da1, db0, db1, w1v, w2v, wsp1, wsp2,
    r1a, r1b, r2a, r2b, aca, acb,
    s1a, s1b, s2a, s2b, sha, shb, soa, sob,
):
    wid = lax.axis_index("s") * NC + lax.axis_index("c")
    bt = wid * (T // NW)
    da = [da0, da1]
    db = [db0, db1]
    r1 = [r1a, r1b]
    r2 = [r2a, r2b]
    ac = [aca, acb]
    s1 = [s1a, s1b]
    s2 = [s2a, s2b]
    sh = [sha, shb]
    so = [soa, sob]
    pltpu.sync_copy(wp_hbm.at[pl.ds(bt, 128)], w1v)
    pltpu.sync_copy(wp_hbm.at[pl.ds(T + bt, 128)], w2v)

    def start_chunk(c):
        b = c % 2
        t0 = bt + c * 16
        pltpu.sync_copy(dest_hbm.at[pl.ds(t0, 16)], da[b])
        pltpu.sync_copy(dest_hbm.at[pl.ds(T + t0, 16)], db[b])
        return (
            pltpu.async_copy(yex_hbm.at[da[b]], r1[b], s1[b]),
            pltpu.async_copy(yex_hbm.at[db[b]], r2[b], s2[b]),
            pltpu.async_copy(ysh_hbm.at[pl.ds(t0, 16)], ac[b], sh[b]),
        )

    descs = [start_chunk(0), start_chunk(1)]
    wdesc = [None, None]
    for c in range(8):
        b = c % 2
        for d in descs[b]:
            d.wait()

        wv1 = w1v[pl.ds(c * 16, 16)]
        wv2 = w2v[pl.ds(c * 16, 16)]
        for r16 in range(16):
            wsp1[r16, :] = jnp.full((NL,), wv1[r16], jnp.float32)
            wsp2[r16, :] = jnp.full((NL,), wv2[r16], jnp.float32)

        def row_body(r, carry):
            w1s = wsp1[r, :]
            w2s = wsp2[r, :]
            for v in range(D // NL):
                sl = pl.ds(v * NL, NL)
                ac[b][r, sl] = ac[b][r, sl] + w1s * r1[b][r, sl] + w2s * r2[b][r, sl]
            return carry

        lax.fori_loop(0, 16, row_body, 0)
        wdesc[b] = pltpu.async_copy(ac[b], out_hbm.at[pl.ds(bt + c * 16, 16)], so[b])
        if c + 2 < 8:
            wdesc[b].wait()
            descs[b] = start_chunk(c + 2)
    wdesc[0].wait()
    wdesc[1].wait()


def _combine(Yex, Ysh, dest, wp_flat):
    mesh = plsc.VectorSubcoreMesh(core_axis_name="c", subcore_axis_name="s")
    f = pl.kernel(
        _combine_body,
        out_type=jax.ShapeDtypeStruct((T, D), jnp.float32),
        mesh=mesh,
        scratch_types=[
            pltpu.VMEM((16,), jnp.int32),
            pltpu.VMEM((16,), jnp.int32),
            pltpu.VMEM((16,), jnp.int32),
            pltpu.VMEM((16,), jnp.int32),
            pltpu.VMEM((128,), jnp.float32),
            pltpu.VMEM((128,), jnp.float32),
            pltpu.VMEM((16, NL), jnp.float32),
            pltpu.VMEM((16, NL), jnp.float32),
            pltpu.VMEM((16, D), jnp.float32),
            pltpu.VMEM((16, D), jnp.float32),
            pltpu.VMEM((16, D), jnp.float32),
            pltpu.VMEM((16, D), jnp.float32),
            pltpu.VMEM((16, D), jnp.float32),
            pltpu.VMEM((16, D), jnp.float32),
            pltpu.SemaphoreType.DMA,
            pltpu.SemaphoreType.DMA,
            pltpu.SemaphoreType.DMA,
            pltpu.SemaphoreType.DMA,
            pltpu.SemaphoreType.DMA,
            pltpu.SemaphoreType.DMA,
            pltpu.SemaphoreType.DMA,
            pltpu.SemaphoreType.DMA,
        ],
    )
    return f(Yex, Ysh, dest, wp_flat)


def kernel(x, Wr, Wg, Wu, Wd, Sg, Su, Sd, expert_bias):
    bsz, seqlen, dim = x.shape
    x_flat = x.reshape(bsz * seqlen, dim)

    dest, wp, be = _route(x_flat, Wr, expert_bias)
    Xs = _dispatch(dest, x_flat)
    Ysh = _shared_expert(x_flat, Sg, Su, Sd)
    Yex = _grouped_gemm(be, Xs, Wg, Wu, Wd)
    out = _combine(Yex, Ysh, dest, wp.reshape(P))
    return out.reshape(bsz, seqlen, dim)
